# baseline (speedup 1.0000x reference)
import functools

import jax
import jax.numpy as jnp
from jax import lax
from jax.experimental import pallas as pl
from jax.experimental.pallas import tpu as pltpu

_U = 8
_TS = 256


def _emb_ln_kernel(ids_ref,
                   tok_hbm,
                   pos_ref,
                   segf_ref,
                   dseg_ref,
                   gamma_ref,
                   beta_ref,
                   o_ref,
                   gbuf,
                   sems):
    TS, D = pos_ref.shape
    core = pl.program_id(0)
    j = pl.program_id(1)
    nj = pl.num_programs(1)
    nb = nj // 2
    slot = lax.rem(j, 2)

    def issue_tile(jj, sl):
        st = jj // nb
        b = core * nb + lax.rem(jj, nb)
        base = b * (2 * TS) + st * TS

        def chunk(k):
            r0 = k * _U
            for u in range(_U):
                r = r0 + u
                idx = ids_ref[base + r]
                pltpu.make_async_copy(
                    tok_hbm.at[pl.ds(idx, 1)],
                    gbuf.at[sl, pl.ds(r, 1)],
                    sems.at[sl]).start(priority=u % 2)

        pl.loop(0, TS // _U)(chunk)

    @pl.when(j == 0)
    def _prime():
        issue_tile(j, 0)

    @pl.when(j + 1 < nj)
    def _prefetch():
        issue_tile(j + 1, 1 - slot)

    pltpu.make_async_copy(tok_hbm.at[pl.ds(0, TS)], gbuf.at[slot],
                          sems.at[slot]).wait()

    emb = gbuf[slot] + pos_ref[...] + segf_ref[0] * dseg_ref[...]
    mean = jnp.mean(emb, axis=-1, keepdims=True)
    cen = emb - mean
    var = jnp.mean(cen * cen, axis=-1, keepdims=True)
    normed = cen * lax.rsqrt(var + 1e-5)
    o_ref[0] = normed * gamma_ref[...] + beta_ref[...]


@functools.partial(jax.jit, static_argnames=())
def kernel(x, seg, tok_tab, pos_tab, seg_tab, gamma, beta):
    B, S = x.shape
    V, D = tok_tab.shape
    assert B % 2 == 0 and S % _TS == 0 and (S // _TS) == 2
    nb = B // 2
    n_st = S // _TS

    ids_flat = jnp.clip(x.reshape(B * S).astype(jnp.int32), 0, V - 1)
    pos2 = pos_tab[:S] + seg_tab[0][None, :]
    dseg = (seg_tab[1] - seg_tab[0]).reshape(1, D)
    segf = seg.reshape(B * n_st, _TS, 1).astype(jnp.float32)
    gamma2 = gamma.reshape(1, D)
    beta2 = beta.reshape(1, D)

    def tile_idx(c, j):
        st = j // nb
        b = c * nb + lax.rem(j, nb)
        return b * n_st + st

    grid_spec = pltpu.PrefetchScalarGridSpec(
        num_scalar_prefetch=1,
        grid=(2, nb * n_st),
        in_specs=[
            pl.BlockSpec(memory_space=pl.ANY),
            pl.BlockSpec((_TS, D), lambda c, j, ids: (j // nb, 0)),
            pl.BlockSpec((1, _TS, 1), lambda c, j, ids: (tile_idx(c, j), 0, 0)),
            pl.BlockSpec((1, D), lambda c, j, ids: (0, 0)),
            pl.BlockSpec((1, D), lambda c, j, ids: (0, 0)),
            pl.BlockSpec((1, D), lambda c, j, ids: (0, 0)),
        ],
        out_specs=pl.BlockSpec((1, _TS, D),
                               lambda c, j, ids: (tile_idx(c, j), 0, 0)),
        scratch_shapes=[
            pltpu.VMEM((2, _TS, D), tok_tab.dtype),
            pltpu.SemaphoreType.DMA((2,)),
        ],
    )

    out = pl.pallas_call(
        _emb_ln_kernel,
        out_shape=jax.ShapeDtypeStruct((B * n_st, _TS, D), jnp.float32),
        grid_spec=grid_spec,
        compiler_params=pltpu.CompilerParams(
            dimension_semantics=("parallel", "arbitrary"),
            disable_bounds_checks=True,
        ),
    )(ids_flat, tok_tab, pos2, segf, dseg, gamma2, beta2)
    return out.reshape(B, S, D)
